# baseline (speedup 1.0000x reference)
import math

import jax
import jax.numpy as jnp
from jax import lax
from jax.experimental import pallas as pl
from jax.experimental.pallas import tpu as pltpu

TOK = 2048
DIM = 768
NE = 8
FFD = 1536
K = 2
CAP = TOK

RT = 256
GT = 256
NPAIR = K * TOK
NUM_TILES = (NPAIR + NE * (GT - 1) + GT - 1) // GT
PADDED = NUM_TILES * GT
TPE = CAP // GT

_SQRT2 = math.sqrt(2.0)


def _router_body(x_ref, gw_ref, pos_ref, wts_ref, stats_ref, cnt_ref):
    i = pl.program_id(0)

    @pl.when(i == 0)
    def _():
        cnt_ref[...] = jnp.zeros_like(cnt_ref)
        stats_ref[...] = jnp.zeros_like(stats_ref)

    x = x_ref[...]
    logits = jnp.dot(x, gw_ref[...], preferred_element_type=jnp.float32)
    col = lax.broadcasted_iota(jnp.int32, logits.shape, 1)
    m1 = jnp.max(logits, axis=1)
    i1 = jnp.argmax(logits, axis=1)
    masked = jnp.where(col == i1[:, None], -jnp.inf, logits)
    m2 = jnp.max(masked, axis=1)
    i2 = jnp.argmax(masked, axis=1)
    z = jnp.exp(m2 - m1)
    wa = 1.0 / (1.0 + z)
    wb = z * wa

    oh1 = (col == i1[:, None]).astype(jnp.float32)
    oh2 = (col == i2[:, None]).astype(jnp.float32)
    r_ = lax.broadcasted_iota(jnp.int32, (RT, RT), 0)
    c_ = lax.broadcasted_iota(jnp.int32, (RT, RT), 1)
    tri = (r_ > c_).astype(jnp.float32)
    c1 = jnp.dot(tri, oh1, preferred_element_type=jnp.float32)
    c2 = jnp.dot(tri, oh2, preferred_element_type=jnp.float32)
    tot1 = jnp.sum(oh1, axis=0, keepdims=True)
    tot2 = jnp.sum(oh2, axis=0, keepdims=True)
    cnt = cnt_ref[...]
    rank1 = jnp.sum((cnt + c1) * oh1, axis=1)
    rank2 = jnp.sum((cnt + tot1 + c2) * oh2, axis=1)
    pos1 = i1 * CAP + rank1.astype(jnp.int32)
    pos2 = i2 * CAP + rank2.astype(jnp.int32)
    pos_ref[...] = jnp.stack([pos1, pos2], axis=0)
    wts_ref[...] = jnp.stack([wa, wb], axis=0)
    cnt_ref[...] = cnt + tot1 + tot2

    probs = jax.nn.softmax(logits, axis=1)
    psum = jnp.sum(probs, axis=0, keepdims=True)
    sq = jnp.sum(logits * logits)
    row = lax.broadcasted_iota(jnp.int32, (8, NE), 0)
    upd = jnp.where(row == 0, tot1,
                    jnp.where(row == 1, psum,
                              jnp.where(row == 2, sq, tot1 + tot2)))
    upd = jnp.where(row >= 4, 0.0, upd)
    stats_ref[...] += upd


def _ffn_body(te_ref, tv_ref, x_ref, w1_ref, w2_ref, b1_ref, b2_ref, o_ref):
    @pl.when(tv_ref[pl.program_id(0)] != 0)
    def _():
        h = jnp.dot(x_ref[...], w1_ref[0], preferred_element_type=jnp.float32)
        h = h + b1_ref[0]
        h = jnp.maximum(h, 0.0)
        o = jnp.dot(h, w2_ref[0], preferred_element_type=jnp.float32)
        o_ref[...] = o + b2_ref[0]


def kernel(x, gate_w, expert_w1, expert_w2, expert_b1, expert_b2):
    xf = x.reshape(TOK, DIM)

    pos, wts, stats = pl.pallas_call(
        _router_body,
        grid=(TOK // RT,),
        in_specs=[
            pl.BlockSpec((RT, DIM), lambda i: (i, 0)),
            pl.BlockSpec((DIM, NE), lambda i: (0, 0)),
        ],
        out_specs=[
            pl.BlockSpec((K, RT), lambda i: (0, i)),
            pl.BlockSpec((K, RT), lambda i: (0, i)),
            pl.BlockSpec((8, NE), lambda i: (0, 0)),
        ],
        out_shape=[
            jax.ShapeDtypeStruct((K, TOK), jnp.int32),
            jax.ShapeDtypeStruct((K, TOK), jnp.float32),
            jax.ShapeDtypeStruct((8, NE), jnp.float32),
        ],
        scratch_shapes=[pltpu.VMEM((1, NE), jnp.float32)],
    )(xf, gate_w)

    cnt1 = stats[0]
    psum = stats[1]
    sq = stats[2, 0]
    paircnt = stats[3].astype(jnp.int32)
    aux_loss = NE * jnp.sum(cnt1 * psum) / (TOK * TOK)
    z_loss = sq / (TOK * NE) * 0.001
    total_aux = aux_loss + z_loss

    ntiles = (paircnt + GT - 1) // GT
    end_t = jnp.cumsum(ntiles)
    start_t = end_t - ntiles
    used = end_t[NE - 1]
    g = jnp.arange(NUM_TILES, dtype=jnp.int32)
    te_raw = jnp.sum((end_t[None, :] <= g[:, None]).astype(jnp.int32), axis=1)
    maxe = jnp.max(jnp.where(paircnt > 0, jnp.arange(NE), 0)).astype(jnp.int32)
    te = jnp.where(g < used, jnp.minimum(te_raw, NE - 1), maxe)
    r = g - start_t[te]
    toff = jnp.where(g < used, te * TPE + r,
                     maxe * TPE + jnp.maximum(ntiles[maxe] - 1, 0))
    tvalid = (g < used).astype(jnp.int32)
    dshift = (jnp.arange(NE, dtype=jnp.int32) * TPE - start_t) * GT
    cpos = pos - dshift[pos // CAP]

    tok_iota = jnp.arange(TOK, dtype=jnp.int32)
    gather_idx = jnp.zeros((PADDED,), jnp.int32)
    gather_idx = gather_idx.at[cpos[0]].set(tok_iota).at[cpos[1]].set(tok_iota)

    x_sorted = jnp.take(xf, gather_idx, axis=0)

    res = pl.pallas_call(
        _ffn_body,
        grid_spec=pltpu.PrefetchScalarGridSpec(
            num_scalar_prefetch=2,
            grid=(NUM_TILES,),
            in_specs=[
                pl.BlockSpec((GT, DIM), lambda i, te, tv: (i, 0)),
                pl.BlockSpec((1, DIM, FFD), lambda i, te, tv: (te[i], 0, 0)),
                pl.BlockSpec((1, FFD, DIM), lambda i, te, tv: (te[i], 0, 0)),
                pl.BlockSpec((1, 1, FFD), lambda i, te, tv: (te[i], 0, 0)),
                pl.BlockSpec((1, 1, DIM), lambda i, te, tv: (te[i], 0, 0)),
            ],
            out_specs=pl.BlockSpec((GT, DIM), lambda i, te, tv: (i, 0)),
        ),
        out_shape=jax.ShapeDtypeStruct((PADDED, DIM), jnp.float32),
    )(te, tvalid, x_sorted, expert_w1, expert_w2,
      expert_b1.reshape(NE, 1, FFD), expert_b2.reshape(NE, 1, DIM))

    out = wts[0][:, None] * jnp.take(res, cpos[0], axis=0) \
        + wts[1][:, None] * jnp.take(res, cpos[1], axis=0)
    return (out.reshape(x.shape), total_aux)


# speedup vs baseline: 1.0562x; 1.0562x over previous
import math

import jax
import jax.numpy as jnp
from jax import lax
from jax.experimental import pallas as pl
from jax.experimental.pallas import tpu as pltpu

TOK = 2048
DIM = 768
NE = 8
FFD = 1536
K = 2
CAP = TOK

RT = 256
GT = 512
NPAIR = K * TOK
TPE = CAP // GT
NSTEP = NE * TPE
NUM_TILES = NPAIR // GT + NE
PADDED = NUM_TILES * GT

_SQRT2 = math.sqrt(2.0)


def _router_body(x_ref, gw_ref, pos_ref, wts_ref, stats_ref, cnt_ref):
    i = pl.program_id(0)

    @pl.when(i == 0)
    def _():
        cnt_ref[...] = jnp.zeros_like(cnt_ref)
        stats_ref[...] = jnp.zeros_like(stats_ref)

    x = x_ref[...]
    logits = jnp.dot(x, gw_ref[...], preferred_element_type=jnp.float32)
    col = lax.broadcasted_iota(jnp.int32, logits.shape, 1)
    m1 = jnp.max(logits, axis=1)
    i1 = jnp.argmax(logits, axis=1)
    masked = jnp.where(col == i1[:, None], -jnp.inf, logits)
    m2 = jnp.max(masked, axis=1)
    i2 = jnp.argmax(masked, axis=1)
    z = jnp.exp(m2 - m1)
    wa = 1.0 / (1.0 + z)
    wb = z * wa

    oh1 = (col == i1[:, None]).astype(jnp.float32)
    oh2 = (col == i2[:, None]).astype(jnp.float32)
    r_ = lax.broadcasted_iota(jnp.int32, (RT, RT), 0)
    c_ = lax.broadcasted_iota(jnp.int32, (RT, RT), 1)
    tri = (r_ > c_).astype(jnp.float32)
    c1 = jnp.dot(tri, oh1, preferred_element_type=jnp.float32)
    c2 = jnp.dot(tri, oh2, preferred_element_type=jnp.float32)
    tot1 = jnp.sum(oh1, axis=0, keepdims=True)
    tot2 = jnp.sum(oh2, axis=0, keepdims=True)
    cnt = cnt_ref[...]
    rank1 = jnp.sum((cnt + c1) * oh1, axis=1)
    rank2 = jnp.sum((cnt + tot1 + c2) * oh2, axis=1)
    pos_ref[...] = jnp.stack([i1 * CAP + rank1.astype(jnp.int32),
                              i2 * CAP + rank2.astype(jnp.int32)], axis=0)
    wts_ref[...] = jnp.stack([wa, wb], axis=0)
    cnt_ref[...] = cnt + tot1 + tot2

    probs = jax.nn.softmax(logits, axis=1)
    psum = jnp.sum(probs, axis=0, keepdims=True)
    sq = jnp.sum(logits * logits)
    row = lax.broadcasted_iota(jnp.int32, (8, NE), 0)
    upd = jnp.where(row == 0, tot1,
                    jnp.where(row == 1, psum,
                              jnp.where(row == 2, sq, tot1 + tot2)))
    upd = jnp.where(row >= 4, 0.0, upd)
    stats_ref[...] += upd


def _meta_body(stats_ref, pos_ref, cpos_ref, toff_ref, tv_ref, aux_ref):
    stats = stats_ref[...]
    row8 = lax.broadcasted_iota(jnp.int32, (8, 1), 0)
    pc_row = jnp.sum(jnp.where(lax.broadcasted_iota(jnp.int32, (8, NE), 0) == 3,
                               stats, 0.0), axis=0, keepdims=True)
    outer = jnp.dot(jnp.ones((8, 1), jnp.float32), pc_row,
                    preferred_element_type=jnp.float32)
    eye = (row8 == lax.broadcasted_iota(jnp.int32, (8, 8), 1) - 0).astype(jnp.float32)
    eye = (lax.broadcasted_iota(jnp.int32, (8, 8), 0)
           == lax.broadcasted_iota(jnp.int32, (8, 8), 1)).astype(jnp.float32)
    pcT = jnp.sum(outer * eye, axis=1, keepdims=True)
    ntT = jnp.floor((pcT + (GT - 1)) / GT)
    ltri = (lax.broadcasted_iota(jnp.int32, (8, 8), 0)
            >= lax.broadcasted_iota(jnp.int32, (8, 8), 1)).astype(jnp.float32)
    endT = jnp.dot(ltri, ntT, preferred_element_type=jnp.float32)
    startT = endT - ntT
    used = jnp.sum(jnp.where(row8 == 7, endT, 0.0))

    m32 = lax.broadcasted_iota(jnp.int32, (1, NSTEP), 1)
    em = m32 // TPE
    jm = (m32 - em * TPE).astype(jnp.float32)
    ohe = (lax.broadcasted_iota(jnp.int32, (8, NSTEP), 0) == em).astype(jnp.float32)
    nt_m = jnp.sum(ohe * ntT, axis=0, keepdims=True)
    st_m = jnp.sum(ohe * startT, axis=0, keepdims=True)
    validm = jm < nt_m
    toffm = jnp.where(validm, st_m + jm, st_m + nt_m - 1.0)
    toffm = jnp.clip(toffm, 0.0, jnp.maximum(used - 1.0, 0.0))
    toff_ref[...] = toffm.astype(jnp.int32)
    tv_ref[...] = validm.astype(jnp.int32)

    pos = pos_ref[...]
    e_p = pos // CAP
    acc = jnp.zeros_like(pos)
    for j in range(NE):
        dsj = jnp.sum(jnp.where(row8 == j, jnp.float32(j * TPE) - startT, 0.0))
        acc = jnp.where(e_p == j, (dsj * GT).astype(jnp.int32), acc)
    cpos_ref[...] = pos - acc

    rowsel = lax.broadcasted_iota(jnp.int32, (8, NE), 0)
    cnt1 = jnp.sum(jnp.where(rowsel == 0, stats, 0.0), axis=0)
    psum = jnp.sum(jnp.where(rowsel == 1, stats, 0.0), axis=0)
    colsel = lax.broadcasted_iota(jnp.int32, (8, NE), 1)
    sq = jnp.sum(jnp.where((rowsel == 2) & (colsel == 0), stats, 0.0))
    aux = NE * jnp.sum(cnt1 * psum) / (TOK * TOK) + sq * 0.001 / (TOK * NE)
    aux_ref[...] = jnp.full((1, 1), 0.0) + aux


def _ffn_body(toff_ref, tv_ref, x_ref, w1_ref, w2_ref, b1_ref, b2_ref, o_ref):
    e = pl.program_id(0)
    j = pl.program_id(1)

    @pl.when(tv_ref[e * TPE + j] != 0)
    def _():
        h = jnp.dot(x_ref[...], w1_ref[0], preferred_element_type=jnp.float32)
        h = h + b1_ref[0]
        h = 0.5 * h * (1.0 + lax.erf(h / _SQRT2))
        o = jnp.dot(h, w2_ref[0], preferred_element_type=jnp.float32)
        o_ref[...] = o + b2_ref[0]


def kernel(x, gate_w, expert_w1, expert_w2, expert_b1, expert_b2):
    xf = x.reshape(TOK, DIM)

    pos, wts, stats = pl.pallas_call(
        _router_body,
        grid=(TOK // RT,),
        in_specs=[
            pl.BlockSpec((RT, DIM), lambda i: (i, 0)),
            pl.BlockSpec((DIM, NE), lambda i: (0, 0)),
        ],
        out_specs=[
            pl.BlockSpec((K, RT), lambda i: (0, i)),
            pl.BlockSpec((K, RT), lambda i: (0, i)),
            pl.BlockSpec((8, NE), lambda i: (0, 0)),
        ],
        out_shape=[
            jax.ShapeDtypeStruct((K, TOK), jnp.int32),
            jax.ShapeDtypeStruct((K, TOK), jnp.float32),
            jax.ShapeDtypeStruct((8, NE), jnp.float32),
        ],
        scratch_shapes=[pltpu.VMEM((1, NE), jnp.float32)],
    )(xf, gate_w)

    cpos, toff, tvalid, aux = pl.pallas_call(
        _meta_body,
        grid=(1,),
        in_specs=[
            pl.BlockSpec((8, NE), lambda i: (0, 0)),
            pl.BlockSpec((K, TOK), lambda i: (0, 0)),
        ],
        out_specs=[
            pl.BlockSpec((K, TOK), lambda i: (0, 0)),
            pl.BlockSpec((1, NSTEP), lambda i: (0, 0)),
            pl.BlockSpec((1, NSTEP), lambda i: (0, 0)),
            pl.BlockSpec((1, 1), lambda i: (0, 0)),
        ],
        out_shape=[
            jax.ShapeDtypeStruct((K, TOK), jnp.int32),
            jax.ShapeDtypeStruct((1, NSTEP), jnp.int32),
            jax.ShapeDtypeStruct((1, NSTEP), jnp.int32),
            jax.ShapeDtypeStruct((1, 1), jnp.float32),
        ],
    )(stats, pos)
    total_aux = aux[0, 0]

    tok_iota = jnp.arange(TOK, dtype=jnp.int32)
    gather_idx = jnp.zeros((PADDED,), jnp.int32)
    gather_idx = gather_idx.at[cpos[0]].set(tok_iota).at[cpos[1]].set(tok_iota)
    x_sorted = jnp.take(xf, gather_idx, axis=0)

    res = pl.pallas_call(
        _ffn_body,
        grid_spec=pltpu.PrefetchScalarGridSpec(
            num_scalar_prefetch=2,
            grid=(NE, TPE),
            in_specs=[
                pl.BlockSpec((GT, DIM), lambda e, j, toff, tv: (toff[e * TPE + j], 0)),
                pl.BlockSpec((1, DIM, FFD), lambda e, j, toff, tv: (e, 0, 0)),
                pl.BlockSpec((1, FFD, DIM), lambda e, j, toff, tv: (e, 0, 0)),
                pl.BlockSpec((1, 1, FFD), lambda e, j, toff, tv: (e, 0, 0)),
                pl.BlockSpec((1, 1, DIM), lambda e, j, toff, tv: (e, 0, 0)),
            ],
            out_specs=pl.BlockSpec((GT, DIM),
                                   lambda e, j, toff, tv: (toff[e * TPE + j], 0)),
        ),
        out_shape=jax.ShapeDtypeStruct((PADDED, DIM), jnp.float32),
    )(toff.reshape(NSTEP), tvalid.reshape(NSTEP), x_sorted, expert_w1, expert_w2,
      expert_b1.reshape(NE, 1, FFD), expert_b2.reshape(NE, 1, DIM))

    out = wts[0][:, None] * jnp.take(res, cpos[0], axis=0) \
        + wts[1][:, None] * jnp.take(res, cpos[1], axis=0)
    return (out.reshape(x.shape), total_aux)


# speedup vs baseline: 1.6107x; 1.5250x over previous
import functools
import math

import jax
import jax.numpy as jnp
from jax import lax
from jax.experimental import pallas as pl
from jax.experimental.pallas import tpu as pltpu
from jax.experimental.pallas import tpu_sc as plsc

TOK = 2048
DIM = 768
NE = 8
FFD = 1536
K = 2
CAP = TOK

RT = 256
GT = 512
NPAIR = K * TOK
TPE = CAP // GT
NSTEP = NE * TPE
NUM_TILES = NPAIR // GT + NE
PADDED = NUM_TILES * GT

_SQRT2 = math.sqrt(2.0)


def _router_body(x_ref, gw_ref, pos_ref, wts_ref, stats_ref, cnt_ref):
    i = pl.program_id(0)

    @pl.when(i == 0)
    def _():
        cnt_ref[...] = jnp.zeros_like(cnt_ref)
        stats_ref[...] = jnp.zeros_like(stats_ref)

    x = x_ref[...]
    logits = jnp.dot(x, gw_ref[...], preferred_element_type=jnp.float32)
    col = lax.broadcasted_iota(jnp.int32, logits.shape, 1)
    m1 = jnp.max(logits, axis=1)
    i1 = jnp.argmax(logits, axis=1)
    masked = jnp.where(col == i1[:, None], -jnp.inf, logits)
    m2 = jnp.max(masked, axis=1)
    i2 = jnp.argmax(masked, axis=1)
    z = jnp.exp(m2 - m1)
    wa = 1.0 / (1.0 + z)
    wb = z * wa

    oh1 = (col == i1[:, None]).astype(jnp.float32)
    oh2 = (col == i2[:, None]).astype(jnp.float32)
    r_ = lax.broadcasted_iota(jnp.int32, (RT, RT), 0)
    c_ = lax.broadcasted_iota(jnp.int32, (RT, RT), 1)
    tri = (r_ > c_).astype(jnp.float32)
    c1 = jnp.dot(tri, oh1, preferred_element_type=jnp.float32)
    c2 = jnp.dot(tri, oh2, preferred_element_type=jnp.float32)
    tot1 = jnp.sum(oh1, axis=0, keepdims=True)
    tot2 = jnp.sum(oh2, axis=0, keepdims=True)
    cnt = cnt_ref[...]
    rank1 = jnp.sum((cnt + c1) * oh1, axis=1)
    rank2 = jnp.sum((cnt + tot1 + c2) * oh2, axis=1)
    pos_ref[...] = jnp.stack([i1 * CAP + rank1.astype(jnp.int32),
                              i2 * CAP + rank2.astype(jnp.int32)], axis=0)
    wts_ref[...] = jnp.stack([wa, wb], axis=0)
    cnt_ref[...] = cnt + tot1 + tot2

    probs = jax.nn.softmax(logits, axis=1)
    psum = jnp.sum(probs, axis=0, keepdims=True)
    sq = jnp.sum(logits * logits)
    row = lax.broadcasted_iota(jnp.int32, (8, NE), 0)
    upd = jnp.where(row == 0, tot1,
                    jnp.where(row == 1, psum,
                              jnp.where(row == 2, sq, tot1 + tot2)))
    upd = jnp.where(row >= 4, 0.0, upd)
    stats_ref[...] += upd


def _meta_body(stats_ref, pos_ref, cpos_ref, toff_ref, tv_ref, aux_ref):
    stats = stats_ref[...]
    row8 = lax.broadcasted_iota(jnp.int32, (8, 1), 0)
    pc_row = jnp.sum(jnp.where(lax.broadcasted_iota(jnp.int32, (8, NE), 0) == 3,
                               stats, 0.0), axis=0, keepdims=True)
    outer = jnp.dot(jnp.ones((8, 1), jnp.float32), pc_row,
                    preferred_element_type=jnp.float32)
    eye = (row8 == lax.broadcasted_iota(jnp.int32, (8, 8), 1) - 0).astype(jnp.float32)
    eye = (lax.broadcasted_iota(jnp.int32, (8, 8), 0)
           == lax.broadcasted_iota(jnp.int32, (8, 8), 1)).astype(jnp.float32)
    pcT = jnp.sum(outer * eye, axis=1, keepdims=True)
    ntT = jnp.floor((pcT + (GT - 1)) / GT)
    ltri = (lax.broadcasted_iota(jnp.int32, (8, 8), 0)
            >= lax.broadcasted_iota(jnp.int32, (8, 8), 1)).astype(jnp.float32)
    endT = jnp.dot(ltri, ntT, preferred_element_type=jnp.float32)
    startT = endT - ntT
    used = jnp.sum(jnp.where(row8 == 7, endT, 0.0))

    m32 = lax.broadcasted_iota(jnp.int32, (1, NSTEP), 1)
    em = m32 // TPE
    jm = (m32 - em * TPE).astype(jnp.float32)
    ohe = (lax.broadcasted_iota(jnp.int32, (8, NSTEP), 0) == em).astype(jnp.float32)
    nt_m = jnp.sum(ohe * ntT, axis=0, keepdims=True)
    st_m = jnp.sum(ohe * startT, axis=0, keepdims=True)
    validm = jm < nt_m
    toffm = jnp.where(validm, st_m + jm, st_m + nt_m - 1.0)
    toffm = jnp.clip(toffm, 0.0, jnp.maximum(used - 1.0, 0.0))
    toff_ref[...] = toffm.astype(jnp.int32)
    tv_ref[...] = validm.astype(jnp.int32)

    pos = pos_ref[...]
    e_p = pos // CAP
    acc = jnp.zeros_like(pos)
    for j in range(NE):
        dsj = jnp.sum(jnp.where(row8 == j, jnp.float32(j * TPE) - startT, 0.0))
        acc = jnp.where(e_p == j, (dsj * GT).astype(jnp.int32), acc)
    cpos_ref[...] = pos - acc

    rowsel = lax.broadcasted_iota(jnp.int32, (8, NE), 0)
    cnt1 = jnp.sum(jnp.where(rowsel == 0, stats, 0.0), axis=0)
    psum = jnp.sum(jnp.where(rowsel == 1, stats, 0.0), axis=0)
    colsel = lax.broadcasted_iota(jnp.int32, (8, NE), 1)
    sq = jnp.sum(jnp.where((rowsel == 2) & (colsel == 0), stats, 0.0))
    aux = NE * jnp.sum(cnt1 * psum) / (TOK * TOK) + sq * 0.001 / (TOK * NE)
    aux_ref[...] = jnp.full((1, 1), 0.0) + aux


NW = 32
PAIR_PER_W = NPAIR // NW
TOK_PER_W = TOK // NW
_SC_MESH = plsc.VectorSubcoreMesh(core_axis_name="c", subcore_axis_name="s")


@functools.partial(
    pl.kernel, mesh=_SC_MESH,
    out_type=jax.ShapeDtypeStruct((PADDED, DIM), jnp.float32),
    scratch_types=[
        pltpu.VMEM((PAIR_PER_W,), jnp.int32),
        pltpu.VMEM((PAIR_PER_W, DIM), jnp.float32),
        pltpu.SemaphoreType.DMA,
    ],
)
def _sc_scatter_x(xf_hbm, cpos_hbm, out_hbm, idx_v, rows_v, sem):
    wid = lax.axis_index("s") * 2 + lax.axis_index("c")
    pltpu.sync_copy(cpos_hbm.at[wid], idx_v)
    src = lax.rem(wid * PAIR_PER_W, TOK)
    pltpu.sync_copy(xf_hbm.at[pl.ds(src, PAIR_PER_W)], rows_v)
    pltpu.async_copy(rows_v, out_hbm.at[idx_v], sem).wait()


@functools.partial(
    pl.kernel, mesh=_SC_MESH,
    out_type=jax.ShapeDtypeStruct((K * TOK, DIM), jnp.float32),
    scratch_types=[
        pltpu.VMEM((TOK_PER_W,), jnp.int32),
        pltpu.VMEM((TOK_PER_W,), jnp.int32),
        pltpu.VMEM((TOK_PER_W, DIM), jnp.float32),
        pltpu.VMEM((TOK_PER_W, DIM), jnp.float32),
        pltpu.SemaphoreType.DMA,
        pltpu.SemaphoreType.DMA,
    ],
)
def _sc_gather2(res_hbm, cpos_hbm, out_hbm, i0_v, i1_v, b0_v, b1_v, s0, s1):
    wid = lax.axis_index("s") * 2 + lax.axis_index("c")
    base = wid * TOK_PER_W
    pltpu.sync_copy(cpos_hbm.at[0].at[pl.ds(base, TOK_PER_W)], i0_v)
    pltpu.sync_copy(cpos_hbm.at[1].at[pl.ds(base, TOK_PER_W)], i1_v)
    c0 = pltpu.async_copy(res_hbm.at[i0_v], b0_v, s0)
    c1 = pltpu.async_copy(res_hbm.at[i1_v], b1_v, s1)
    c0.wait()
    c1.wait()
    pltpu.sync_copy(b0_v, out_hbm.at[pl.ds(base, TOK_PER_W)])
    pltpu.sync_copy(b1_v, out_hbm.at[pl.ds(TOK + base, TOK_PER_W)])


def _ffn_body(toff_ref, tv_ref, x_ref, w1_ref, w2_ref, b1_ref, b2_ref, o_ref):
    e = pl.program_id(0)
    j = pl.program_id(1)

    @pl.when(tv_ref[e * TPE + j] != 0)
    def _():
        h = jnp.dot(x_ref[...], w1_ref[0], preferred_element_type=jnp.float32)
        h = h + b1_ref[0]
        h = 0.5 * h * (1.0 + lax.erf(h / _SQRT2))
        o = jnp.dot(h, w2_ref[0], preferred_element_type=jnp.float32)
        o_ref[...] = o + b2_ref[0]


def kernel(x, gate_w, expert_w1, expert_w2, expert_b1, expert_b2):
    xf = x.reshape(TOK, DIM)

    pos, wts, stats = pl.pallas_call(
        _router_body,
        grid=(TOK // RT,),
        in_specs=[
            pl.BlockSpec((RT, DIM), lambda i: (i, 0)),
            pl.BlockSpec((DIM, NE), lambda i: (0, 0)),
        ],
        out_specs=[
            pl.BlockSpec((K, RT), lambda i: (0, i)),
            pl.BlockSpec((K, RT), lambda i: (0, i)),
            pl.BlockSpec((8, NE), lambda i: (0, 0)),
        ],
        out_shape=[
            jax.ShapeDtypeStruct((K, TOK), jnp.int32),
            jax.ShapeDtypeStruct((K, TOK), jnp.float32),
            jax.ShapeDtypeStruct((8, NE), jnp.float32),
        ],
        scratch_shapes=[pltpu.VMEM((1, NE), jnp.float32)],
    )(xf, gate_w)

    cpos, toff, tvalid, aux = pl.pallas_call(
        _meta_body,
        grid=(1,),
        in_specs=[
            pl.BlockSpec((8, NE), lambda i: (0, 0)),
            pl.BlockSpec((K, TOK), lambda i: (0, 0)),
        ],
        out_specs=[
            pl.BlockSpec((K, TOK), lambda i: (0, 0)),
            pl.BlockSpec((1, NSTEP), lambda i: (0, 0)),
            pl.BlockSpec((1, NSTEP), lambda i: (0, 0)),
            pl.BlockSpec((1, 1), lambda i: (0, 0)),
        ],
        out_shape=[
            jax.ShapeDtypeStruct((K, TOK), jnp.int32),
            jax.ShapeDtypeStruct((1, NSTEP), jnp.int32),
            jax.ShapeDtypeStruct((1, NSTEP), jnp.int32),
            jax.ShapeDtypeStruct((1, 1), jnp.float32),
        ],
    )(stats, pos)
    total_aux = aux[0, 0]

    x_sorted = _sc_scatter_x(xf, cpos.reshape(NW, PAIR_PER_W))

    res = pl.pallas_call(
        _ffn_body,
        grid_spec=pltpu.PrefetchScalarGridSpec(
            num_scalar_prefetch=2,
            grid=(NE, TPE),
            in_specs=[
                pl.BlockSpec((GT, DIM), lambda e, j, toff, tv: (toff[e * TPE + j], 0)),
                pl.BlockSpec((1, DIM, FFD), lambda e, j, toff, tv: (e, 0, 0)),
                pl.BlockSpec((1, FFD, DIM), lambda e, j, toff, tv: (e, 0, 0)),
                pl.BlockSpec((1, 1, FFD), lambda e, j, toff, tv: (e, 0, 0)),
                pl.BlockSpec((1, 1, DIM), lambda e, j, toff, tv: (e, 0, 0)),
            ],
            out_specs=pl.BlockSpec((GT, DIM),
                                   lambda e, j, toff, tv: (toff[e * TPE + j], 0)),
        ),
        out_shape=jax.ShapeDtypeStruct((PADDED, DIM), jnp.float32),
    )(toff.reshape(NSTEP), tvalid.reshape(NSTEP), x_sorted, expert_w1, expert_w2,
      expert_b1.reshape(NE, 1, FFD), expert_b2.reshape(NE, 1, DIM))

    sg = _sc_gather2(res, cpos)
    out = wts[0][:, None] * sg[:TOK] + wts[1][:, None] * sg[TOK:]
    return (out.reshape(x.shape), total_aux)


# speedup vs baseline: 1.6438x; 1.0206x over previous
import functools
import math

import jax
import jax.numpy as jnp
from jax import lax
from jax.experimental import pallas as pl
from jax.experimental.pallas import tpu as pltpu
from jax.experimental.pallas import tpu_sc as plsc

TOK = 2048
DIM = 768
NE = 8
FFD = 1536
K = 2
CAP = TOK

RT = 256
GT = 256
NPAIR = K * TOK
TPE = CAP // GT
NSTEP = NE * TPE
NUM_TILES = NPAIR // GT + NE
PADDED = NUM_TILES * GT

_SQRT2 = math.sqrt(2.0)


def _router_body(x_ref, gw_ref, pos_ref, wts_ref, stats_ref, cnt_ref):
    i = pl.program_id(0)

    @pl.when(i == 0)
    def _():
        cnt_ref[...] = jnp.zeros_like(cnt_ref)
        stats_ref[...] = jnp.zeros_like(stats_ref)

    x = x_ref[...]
    logits = jnp.dot(x, gw_ref[...], preferred_element_type=jnp.float32)
    col = lax.broadcasted_iota(jnp.int32, logits.shape, 1)
    m1 = jnp.max(logits, axis=1)
    i1 = jnp.argmax(logits, axis=1)
    masked = jnp.where(col == i1[:, None], -jnp.inf, logits)
    m2 = jnp.max(masked, axis=1)
    i2 = jnp.argmax(masked, axis=1)
    z = jnp.exp(m2 - m1)
    wa = 1.0 / (1.0 + z)
    wb = z * wa

    oh1 = (col == i1[:, None]).astype(jnp.float32)
    oh2 = (col == i2[:, None]).astype(jnp.float32)
    r_ = lax.broadcasted_iota(jnp.int32, (RT, RT), 0)
    c_ = lax.broadcasted_iota(jnp.int32, (RT, RT), 1)
    tri = (r_ > c_).astype(jnp.float32)
    c1 = jnp.dot(tri, oh1, preferred_element_type=jnp.float32)
    c2 = jnp.dot(tri, oh2, preferred_element_type=jnp.float32)
    tot1 = jnp.sum(oh1, axis=0, keepdims=True)
    tot2 = jnp.sum(oh2, axis=0, keepdims=True)
    cnt = cnt_ref[...]
    rank1 = jnp.sum((cnt + c1) * oh1, axis=1)
    rank2 = jnp.sum((cnt + tot1 + c2) * oh2, axis=1)
    pos_ref[...] = jnp.stack([i1 * CAP + rank1.astype(jnp.int32),
                              i2 * CAP + rank2.astype(jnp.int32)], axis=0)
    wts_ref[...] = jnp.stack([wa, wb], axis=0)
    cnt_ref[...] = cnt + tot1 + tot2

    probs = jax.nn.softmax(logits, axis=1)
    psum = jnp.sum(probs, axis=0, keepdims=True)
    sq = jnp.sum(logits * logits)
    row = lax.broadcasted_iota(jnp.int32, (8, NE), 0)
    upd = jnp.where(row == 0, tot1,
                    jnp.where(row == 1, psum,
                              jnp.where(row == 2, sq, tot1 + tot2)))
    upd = jnp.where(row >= 4, 0.0, upd)
    stats_ref[...] += upd


def _meta_body(stats_ref, pos_ref, cpos_ref, toff_ref, tv_ref, aux_ref):
    stats = stats_ref[...]
    row8 = lax.broadcasted_iota(jnp.int32, (8, 1), 0)
    pc_row = jnp.sum(jnp.where(lax.broadcasted_iota(jnp.int32, (8, NE), 0) == 3,
                               stats, 0.0), axis=0, keepdims=True)
    outer = jnp.dot(jnp.ones((8, 1), jnp.float32), pc_row,
                    preferred_element_type=jnp.float32)
    eye = (row8 == lax.broadcasted_iota(jnp.int32, (8, 8), 1) - 0).astype(jnp.float32)
    eye = (lax.broadcasted_iota(jnp.int32, (8, 8), 0)
           == lax.broadcasted_iota(jnp.int32, (8, 8), 1)).astype(jnp.float32)
    pcT = jnp.sum(outer * eye, axis=1, keepdims=True)
    ntT = jnp.floor((pcT + (GT - 1)) / GT)
    ltri = (lax.broadcasted_iota(jnp.int32, (8, 8), 0)
            >= lax.broadcasted_iota(jnp.int32, (8, 8), 1)).astype(jnp.float32)
    endT = jnp.dot(ltri, ntT, preferred_element_type=jnp.float32)
    startT = endT - ntT
    used = jnp.sum(jnp.where(row8 == 7, endT, 0.0))

    m32 = lax.broadcasted_iota(jnp.int32, (1, NSTEP), 1)
    em = m32 // TPE
    jm = (m32 - em * TPE).astype(jnp.float32)
    ohe = (lax.broadcasted_iota(jnp.int32, (8, NSTEP), 0) == em).astype(jnp.float32)
    nt_m = jnp.sum(ohe * ntT, axis=0, keepdims=True)
    st_m = jnp.sum(ohe * startT, axis=0, keepdims=True)
    validm = jm < nt_m
    toffm = jnp.where(validm, st_m + jm, st_m + nt_m - 1.0)
    toffm = jnp.clip(toffm, 0.0, jnp.maximum(used - 1.0, 0.0))
    toff_ref[...] = toffm.astype(jnp.int32)
    tv_ref[...] = validm.astype(jnp.int32)

    pos = pos_ref[...]
    e_p = pos // CAP
    acc = jnp.zeros_like(pos)
    for j in range(NE):
        dsj = jnp.sum(jnp.where(row8 == j, jnp.float32(j * TPE) - startT, 0.0))
        acc = jnp.where(e_p == j, (dsj * GT).astype(jnp.int32), acc)
    cpos_ref[...] = pos - acc

    rowsel = lax.broadcasted_iota(jnp.int32, (8, NE), 0)
    cnt1 = jnp.sum(jnp.where(rowsel == 0, stats, 0.0), axis=0)
    psum = jnp.sum(jnp.where(rowsel == 1, stats, 0.0), axis=0)
    colsel = lax.broadcasted_iota(jnp.int32, (8, NE), 1)
    sq = jnp.sum(jnp.where((rowsel == 2) & (colsel == 0), stats, 0.0))
    aux = NE * jnp.sum(cnt1 * psum) / (TOK * TOK) + sq * 0.001 / (TOK * NE)
    aux_ref[...] = jnp.full((1, 1), 0.0) + aux


NW = 32
PAIR_PER_W = NPAIR // NW
TOK_PER_W = TOK // NW
_SC_MESH = plsc.VectorSubcoreMesh(core_axis_name="c", subcore_axis_name="s")


@functools.partial(
    pl.kernel, mesh=_SC_MESH,
    out_type=jax.ShapeDtypeStruct((PADDED, DIM), jnp.float32),
    scratch_types=[
        pltpu.VMEM((PAIR_PER_W,), jnp.int32),
        pltpu.VMEM((PAIR_PER_W, DIM), jnp.float32),
        pltpu.SemaphoreType.DMA,
    ],
)
def _sc_scatter_x(xf_hbm, cpos_hbm, out_hbm, idx_v, rows_v, sem):
    wid = lax.axis_index("s") * 2 + lax.axis_index("c")
    pltpu.sync_copy(cpos_hbm.at[wid], idx_v)
    src = lax.rem(wid * PAIR_PER_W, TOK)
    pltpu.sync_copy(xf_hbm.at[pl.ds(src, PAIR_PER_W)], rows_v)
    pltpu.async_copy(rows_v, out_hbm.at[idx_v], sem).wait()


@functools.partial(
    pl.kernel, mesh=_SC_MESH,
    out_type=jax.ShapeDtypeStruct((K * TOK, DIM), jnp.float32),
    scratch_types=[
        pltpu.VMEM((TOK_PER_W,), jnp.int32),
        pltpu.VMEM((TOK_PER_W,), jnp.int32),
        pltpu.VMEM((TOK_PER_W, DIM), jnp.float32),
        pltpu.VMEM((TOK_PER_W, DIM), jnp.float32),
        pltpu.SemaphoreType.DMA,
        pltpu.SemaphoreType.DMA,
    ],
)
def _sc_gather2(res_hbm, cpos_hbm, out_hbm, i0_v, i1_v, b0_v, b1_v, s0, s1):
    wid = lax.axis_index("s") * 2 + lax.axis_index("c")
    base = wid * TOK_PER_W
    pltpu.sync_copy(cpos_hbm.at[0].at[pl.ds(base, TOK_PER_W)], i0_v)
    pltpu.sync_copy(cpos_hbm.at[1].at[pl.ds(base, TOK_PER_W)], i1_v)
    c0 = pltpu.async_copy(res_hbm.at[i0_v], b0_v, s0)
    c1 = pltpu.async_copy(res_hbm.at[i1_v], b1_v, s1)
    c0.wait()
    c1.wait()
    pltpu.sync_copy(b0_v, out_hbm.at[pl.ds(base, TOK_PER_W)])
    pltpu.sync_copy(b1_v, out_hbm.at[pl.ds(TOK + base, TOK_PER_W)])


def _ffn_body(toff_ref, tv_ref, x_ref, w1_ref, w2_ref, b1_ref, b2_ref, o_ref):
    e = pl.program_id(0)
    j = pl.program_id(1)

    @pl.when(tv_ref[e * TPE + j] != 0)
    def _():
        h = jnp.dot(x_ref[...], w1_ref[0], preferred_element_type=jnp.float32)
        h = h + b1_ref[0]
        h = 0.5 * h * (1.0 + lax.erf(h / _SQRT2))
        o = jnp.dot(h, w2_ref[0], preferred_element_type=jnp.float32)
        o_ref[...] = o + b2_ref[0]


def kernel(x, gate_w, expert_w1, expert_w2, expert_b1, expert_b2):
    xf = x.reshape(TOK, DIM)

    pos, wts, stats = pl.pallas_call(
        _router_body,
        grid=(TOK // RT,),
        in_specs=[
            pl.BlockSpec((RT, DIM), lambda i: (i, 0)),
            pl.BlockSpec((DIM, NE), lambda i: (0, 0)),
        ],
        out_specs=[
            pl.BlockSpec((K, RT), lambda i: (0, i)),
            pl.BlockSpec((K, RT), lambda i: (0, i)),
            pl.BlockSpec((8, NE), lambda i: (0, 0)),
        ],
        out_shape=[
            jax.ShapeDtypeStruct((K, TOK), jnp.int32),
            jax.ShapeDtypeStruct((K, TOK), jnp.float32),
            jax.ShapeDtypeStruct((8, NE), jnp.float32),
        ],
        scratch_shapes=[pltpu.VMEM((1, NE), jnp.float32)],
    )(xf, gate_w)

    cpos, toff, tvalid, aux = pl.pallas_call(
        _meta_body,
        grid=(1,),
        in_specs=[
            pl.BlockSpec((8, NE), lambda i: (0, 0)),
            pl.BlockSpec((K, TOK), lambda i: (0, 0)),
        ],
        out_specs=[
            pl.BlockSpec((K, TOK), lambda i: (0, 0)),
            pl.BlockSpec((1, NSTEP), lambda i: (0, 0)),
            pl.BlockSpec((1, NSTEP), lambda i: (0, 0)),
            pl.BlockSpec((1, 1), lambda i: (0, 0)),
        ],
        out_shape=[
            jax.ShapeDtypeStruct((K, TOK), jnp.int32),
            jax.ShapeDtypeStruct((1, NSTEP), jnp.int32),
            jax.ShapeDtypeStruct((1, NSTEP), jnp.int32),
            jax.ShapeDtypeStruct((1, 1), jnp.float32),
        ],
    )(stats, pos)
    total_aux = aux[0, 0]

    x_sorted = _sc_scatter_x(xf, cpos.reshape(NW, PAIR_PER_W))

    res = pl.pallas_call(
        _ffn_body,
        grid_spec=pltpu.PrefetchScalarGridSpec(
            num_scalar_prefetch=2,
            grid=(NE, TPE),
            in_specs=[
                pl.BlockSpec((GT, DIM), lambda e, j, toff, tv: (toff[e * TPE + j], 0)),
                pl.BlockSpec((1, DIM, FFD), lambda e, j, toff, tv: (e, 0, 0)),
                pl.BlockSpec((1, FFD, DIM), lambda e, j, toff, tv: (e, 0, 0)),
                pl.BlockSpec((1, 1, FFD), lambda e, j, toff, tv: (e, 0, 0)),
                pl.BlockSpec((1, 1, DIM), lambda e, j, toff, tv: (e, 0, 0)),
            ],
            out_specs=pl.BlockSpec((GT, DIM),
                                   lambda e, j, toff, tv: (toff[e * TPE + j], 0)),
        ),
        out_shape=jax.ShapeDtypeStruct((PADDED, DIM), jnp.float32),
    )(toff.reshape(NSTEP), tvalid.reshape(NSTEP), x_sorted, expert_w1, expert_w2,
      expert_b1.reshape(NE, 1, FFD), expert_b2.reshape(NE, 1, DIM))

    sg = _sc_gather2(res, cpos)
    out = wts[0][:, None] * sg[:TOK] + wts[1][:, None] * sg[TOK:]
    return (out.reshape(x.shape), total_aux)
